# Initial kernel scaffold; baseline (speedup 1.0000x reference)
import jax
import jax.numpy as jnp
from jax.experimental import pallas as pl


def kernel(images, tgt_in, params):
    raise NotImplementedError("write your pallas kernel here")


# speedup vs baseline: 1.2131x; 1.2131x over previous
import functools

import jax
import jax.numpy as jnp
from jax import lax
from jax.experimental import pallas as pl
from jax.experimental.pallas import tpu as pltpu

F32 = jnp.float32
_EPS = 1e-3


def _shift_w(xr, zrow):
    up = jnp.concatenate([zrow, xr[:-1]], axis=0)
    dn = jnp.concatenate([xr[1:], zrow], axis=0)
    return up, dn


def _conv_kernel(x_ref, w_ref, bss_ref, o_ref, *, H, W, Ci, Co, ph, pw):
    ho = pl.program_id(1)
    zrow = jnp.zeros((1, Ci), F32)
    zs = []
    for r in range(ph):
        acc = None
        for dy in (-1, 0, 1):
            src = ho * ph + r + dy
            srcc = jnp.clip(src, 0, H - 1)
            m = jnp.where(jnp.logical_and(src >= 0, src < H), 1.0, 0.0)
            xr = x_ref[0, srcc] * m
            up, dn = _shift_w(xr, zrow)
            xcat = jnp.concatenate([up, xr, dn], axis=1)
            c = jnp.dot(xcat, w_ref[dy + 1], preferred_element_type=F32)
            acc = c if acc is None else acc + c
        z = jnp.maximum(acc + bss_ref[0:1, :], 0.0)
        z = z * bss_ref[1:2, :] + bss_ref[2:3, :]
        zs.append(z)
    z = zs[0] if ph == 1 else jnp.maximum(zs[0], zs[1])
    if pw > 1:
        z = jnp.max(z.reshape(W // pw, pw, Co), axis=1)
    o_ref[0, 0] = z


def _conv_call(x, w, bss, *, H, W, Ci, Co, ph, pw, B=8):
    Ho, Wo = H // ph, W // pw
    kfn = functools.partial(_conv_kernel, H=H, W=W, Ci=Ci, Co=Co, ph=ph, pw=pw)
    return pl.pallas_call(
        kfn,
        grid=(B, Ho),
        in_specs=[
            pl.BlockSpec((1, H, W, Ci), lambda b, ho: (b, 0, 0, 0)),
            pl.BlockSpec((3, 3 * Ci, Co), lambda b, ho: (0, 0, 0)),
            pl.BlockSpec((3, Co), lambda b, ho: (0, 0)),
        ],
        out_specs=pl.BlockSpec((1, 1, Wo, Co), lambda b, ho: (b, ho, 0, 0)),
        out_shape=jax.ShapeDtypeStruct((B, Ho, Wo, Co), F32),
        compiler_params=pltpu.CompilerParams(
            dimension_semantics=("parallel", "parallel"),
            vmem_limit_bytes=100 * 1024 * 1024,
        ),
    )(x, w.reshape(3, 3 * Ci, Co), bss)


def _conv1_kernel(x_ref, w_ref, bss_ref, o_ref, *, H, W, Co):
    ho = pl.program_id(1)
    zx = jnp.zeros((1, 1), F32)
    zs = []
    for r in range(2):
        rows = []
        for dy in (-1, 0, 1):
            src = ho * 2 + r + dy
            srcc = jnp.clip(src, 0, H - 1)
            m = jnp.where(jnp.logical_and(src >= 0, src < H), 1.0, 0.0)
            xr = x_ref[0, srcc] * m
            left = jnp.concatenate([zx, xr[:, :-1]], axis=1)
            right = jnp.concatenate([xr[:, 1:], zx], axis=1)
            rows += [left, xr, right]
        xcat = jnp.concatenate(rows, axis=0)
        acc = lax.dot_general(xcat, w_ref[...], (((0,), (0,)), ((), ())),
                              preferred_element_type=F32)
        z = jnp.maximum(acc + bss_ref[0:1, :], 0.0)
        z = z * bss_ref[1:2, :] + bss_ref[2:3, :]
        zs.append(jnp.max(z.reshape(W // 2, 2, Co), axis=1))
    o_ref[0, 0] = jnp.maximum(zs[0], zs[1])


def _conv1_call(x, w, bss, *, H=128, W=512, Co=64, B=8):
    kfn = functools.partial(_conv1_kernel, H=H, W=W, Co=Co)
    return pl.pallas_call(
        kfn,
        grid=(B, H // 2),
        in_specs=[
            pl.BlockSpec((1, H, 1, W), lambda b, ho: (b, 0, 0, 0)),
            pl.BlockSpec((9, Co), lambda b, ho: (0, 0)),
            pl.BlockSpec((3, Co), lambda b, ho: (0, 0)),
        ],
        out_specs=pl.BlockSpec((1, 1, W // 2, Co), lambda b, ho: (b, ho, 0, 0)),
        out_shape=jax.ShapeDtypeStruct((B, H // 2, W // 2, Co), F32),
        compiler_params=pltpu.CompilerParams(
            dimension_semantics=("parallel", "parallel"),
            vmem_limit_bytes=100 * 1024 * 1024,
        ),
    )(x, w.reshape(9, Co), bss)


def _bss(p):
    scale = p["gamma"] * lax.rsqrt(p["var"] + _EPS)
    shift = p["beta"] - p["mean"] * scale
    return jnp.stack([p["b"], scale, shift])


def _rowgru_kernel(x_ref, pw_ref, pb_ref, W_ref, U_ref, b_ref, o_ref, gx_ref):
    d = pl.program_id(0)
    Wd = W_ref[d]
    Ud = U_ref[d]
    Wc = jnp.dot(pw_ref[...], Wd, preferred_element_type=F32)
    brow = jnp.dot(pb_ref[...], Wd, preferred_element_type=F32) + b_ref[d, 0:1]
    xflat = x_ref[...].reshape(64 * 128, 256)
    gx = jnp.dot(xflat, Wc, preferred_element_type=F32) + brow
    gx_ref[...] = gx.reshape(64, 128, 384)
    bh = b_ref[d, 1:2]

    def step(i, h):
        t = jnp.where(d == 0, i, 63 - i)
        gxt = gx_ref[t]
        gh = jnp.dot(h, Ud, preferred_element_type=F32) + bh
        z = jax.nn.sigmoid(gxt[:, :128] + gh[:, :128])
        r = jax.nn.sigmoid(gxt[:, 128:256] + gh[:, 128:256])
        cand = jnp.tanh(gxt[:, 256:] + r * gh[:, 256:])
        hn = z * h + (1.0 - z) * cand
        o_ref[0, t] = hn
        return hn

    lax.fori_loop(0, 64, step, jnp.zeros((128, 128), F32))


def _rowgru_call(xr, pw, pb, Wb, Ub, bb):
    return pl.pallas_call(
        _rowgru_kernel,
        grid=(2,),
        in_specs=[
            pl.BlockSpec((64, 128, 256), lambda d: (0, 0, 0)),
            pl.BlockSpec((256, 256), lambda d: (0, 0)),
            pl.BlockSpec((1, 256), lambda d: (0, 0)),
            pl.BlockSpec((2, 256, 384), lambda d: (0, 0, 0)),
            pl.BlockSpec((2, 128, 384), lambda d: (0, 0, 0)),
            pl.BlockSpec((2, 2, 384), lambda d: (0, 0, 0)),
        ],
        out_specs=pl.BlockSpec((1, 64, 128, 128), lambda d: (d, 0, 0, 0)),
        out_shape=jax.ShapeDtypeStruct((2, 64, 128, 128), F32),
        scratch_shapes=[pltpu.VMEM((64, 128, 384), F32)],
        compiler_params=pltpu.CompilerParams(
            dimension_semantics=("parallel",),
            vmem_limit_bytes=110 * 1024 * 1024,
        ),
    )(xr, pw, pb, Wb, Ub, bb)


def _decoder_kernel(encp_ref, encT_ref, onehot_ref, E_ref, W1T_ref, qb_ref,
                    W2_ref, V_ref, Wde_ref, Wdc_ref, db_ref, U_ref,
                    fc1h_ref, fc1c_ref, fc1b_ref, fc2_ref, fc2b_ref,
                    o_ref, w1ft_ref, etop_ref, gxe_ref):
    for b in range(8):
        w1ft_ref[b] = jnp.dot(W1T_ref[...], encT_ref[b],
                              preferred_element_type=F32)
    etop_ref[...] = jnp.dot(E_ref[...], Wde_ref[...],
                            preferred_element_type=F32)
    gxe = jnp.dot(onehot_ref[...], etop_ref[...],
                  preferred_element_type=F32) + db_ref[0:1]
    gxe_ref[...] = gxe.reshape(64, 8, 768)
    bh = db_ref[1:2]
    Vcol = V_ref[...]

    def step(t, h):
        q = jnp.dot(h, W2_ref[...], preferred_element_type=F32)
        qT = jnp.transpose(q) + qb_ref[...]
        srows = []
        for b in range(8):
            tmp = jnp.tanh(w1ft_ref[b] + qT[:, b:b + 1]) * Vcol
            srows.append(jnp.sum(tmp, axis=0, keepdims=True))
        scores = jnp.concatenate(srows, axis=0)
        mx = jnp.max(scores, axis=1, keepdims=True)
        e = jnp.exp(scores - mx)
        den = jnp.sum(e, axis=1, keepdims=True)
        crows = []
        for b in range(8):
            crows.append(jnp.dot(e[b:b + 1], encp_ref[b],
                                 preferred_element_type=F32))
        ctx = jnp.concatenate(crows, axis=0) / den
        gx = gxe_ref[t] + jnp.dot(ctx, Wdc_ref[...],
                                  preferred_element_type=F32)
        gh = jnp.dot(h, U_ref[...], preferred_element_type=F32) + bh
        z = jax.nn.sigmoid(gx[:, :256] + gh[:, :256])
        r = jax.nn.sigmoid(gx[:, 256:512] + gh[:, 256:512])
        cand = jnp.tanh(gx[:, 512:] + r * gh[:, 512:])
        hn = z * h + (1.0 - z) * cand
        pre = jnp.tanh(jnp.dot(hn, fc1h_ref[...], preferred_element_type=F32)
                       + jnp.dot(ctx, fc1c_ref[...], preferred_element_type=F32)
                       + fc1b_ref[...])
        o_ref[t] = (jnp.dot(pre, fc2_ref[...], preferred_element_type=F32)
                    + fc2b_ref[...])
        return hn

    lax.fori_loop(0, 64, step, jnp.zeros((8, 256), F32))


def _decoder_call(encp, encT, onehot, E, W1T, qb, W2, V, Wde, Wdc, db, U,
                  fc1h, fc1c, fc1b, fc2, fc2b):
    return pl.pallas_call(
        _decoder_kernel,
        out_shape=jax.ShapeDtypeStruct((64, 8, 1000), F32),
        scratch_shapes=[
            pltpu.VMEM((8, 256, 1024), F32),
            pltpu.VMEM((1000, 768), F32),
            pltpu.VMEM((64, 8, 768), F32),
        ],
        compiler_params=pltpu.CompilerParams(
            vmem_limit_bytes=120 * 1024 * 1024,
        ),
    )(encp, encT, onehot, E, W1T, qb, W2, V, Wde, Wdc, db, U,
      fc1h, fc1c, fc1b, fc2, fc2b)


def kernel(images, tgt_in, params):
    p = params
    B = images.shape[0]

    x = images.reshape(B, 128, 1, 512)
    x = _conv1_call(x, p["conv1"]["w"], _bss(p["conv1"]))
    x = _conv_call(x, p["conv2"]["w"], _bss(p["conv2"]),
                   H=64, W=256, Ci=64, Co=128, ph=2, pw=2)
    x = _conv_call(x, p["conv3a"]["w"], _bss(p["conv3a"]),
                   H=32, W=128, Ci=128, Co=256, ph=1, pw=1)
    x = _conv_call(x, p["conv3b"]["w"], _bss(p["conv3b"]),
                   H=32, W=128, Ci=256, Co=256, ph=2, pw=1)
    x = _conv_call(x, p["conv4a"]["w"], _bss(p["conv4a"]),
                   H=16, W=128, Ci=256, Co=256, ph=1, pw=1)
    x = _conv_call(x, p["conv4b"]["w"], _bss(p["conv4b"]),
                   H=16, W=128, Ci=256, Co=256, ph=1, pw=2)

    xr = x.transpose(2, 0, 1, 3).reshape(64, 128, 256)
    Wb = jnp.stack([p["row_fwd"]["W"], p["row_bwd"]["W"]])
    Ub = jnp.stack([p["row_fwd"]["U"], p["row_bwd"]["U"]])
    bb = jnp.stack([p["row_fwd"]["b"], p["row_bwd"]["b"]])
    hs = _rowgru_call(xr, p["proj"]["w"], p["proj"]["b"].reshape(1, 256),
                      Wb, Ub, bb)
    enc_t = jnp.concatenate([hs[0], hs[1]], axis=-1)
    encp = enc_t.reshape(64, B, 16, 256).transpose(1, 2, 0, 3).reshape(B, 1024, 256)
    encT = encp.transpose(0, 2, 1)

    onehot = jax.nn.one_hot(tgt_in.T.reshape(-1), 1000, dtype=F32)
    qb = (p["attn_W1"]["b"] + p["attn_W2"]["b"]).reshape(256, 1)
    logits3 = _decoder_call(
        encp, encT, onehot, p["embed"],
        p["attn_W1"]["w"].T, qb, p["attn_W2"]["w"], p["attn_V"]["w"],
        p["dec_gru"]["W"][:128], p["dec_gru"]["W"][128:], p["dec_gru"]["b"],
        p["dec_gru"]["U"],
        p["fc1"]["w"][:256], p["fc1"]["w"][256:], p["fc1"]["b"].reshape(1, 256),
        p["fc2"]["w"], p["fc2"]["b"].reshape(1, 1000))
    return logits3.transpose(1, 0, 2)


# speedup vs baseline: 1.3406x; 1.1052x over previous
import functools

import jax
import jax.numpy as jnp
from jax import lax
from jax.experimental import pallas as pl
from jax.experimental.pallas import tpu as pltpu

F32 = jnp.float32
_EPS = 1e-3


def _conv_kernel(x_ref, w_ref, bss_ref, o_ref, xcat_ref, *,
                 H, W, Ci, Co, ph, pw, pwin):
    Wp = W // pwin
    zrow = jnp.zeros((1, Ci), F32)
    xcat_ref[0] = jnp.zeros((Wp, 3 * Ci), F32)
    xcat_ref[H + 1] = jnp.zeros((Wp, 3 * Ci), F32)

    def build(h, c):
        xr = x_ref[0, h]
        if pwin > 1:
            xr = jnp.max(xr.reshape(Wp, pwin, Ci), axis=1)
        up = jnp.concatenate([zrow, xr[:-1]], axis=0)
        dn = jnp.concatenate([xr[1:], zrow], axis=0)
        xcat_ref[h + 1] = jnp.concatenate([up, xr, dn], axis=1)
        return c

    lax.fori_loop(0, H, build, 0)

    def consume(i, c):
        zs = []
        for r in range(ph):
            cr = ph * i + r
            acc = (jnp.dot(xcat_ref[cr], w_ref[0], preferred_element_type=F32)
                   + jnp.dot(xcat_ref[cr + 1], w_ref[1],
                             preferred_element_type=F32)
                   + jnp.dot(xcat_ref[cr + 2], w_ref[2],
                             preferred_element_type=F32))
            z = jnp.maximum(acc + bss_ref[0:1, :], 0.0)
            zs.append(z * bss_ref[1:2, :] + bss_ref[2:3, :])
        z = zs[0] if ph == 1 else jnp.maximum(zs[0], zs[1])
        if pw > 1:
            z = jnp.max(z.reshape(Wp // pw, pw, Co), axis=1)
        o_ref[0, i] = z
        return c

    lax.fori_loop(0, H // ph, consume, 0)


def _conv_call(x, w, bss, *, H, W, Ci, Co, ph, pw, pwin=1, B=8):
    Wp = W // pwin
    kfn = functools.partial(_conv_kernel, H=H, W=W, Ci=Ci, Co=Co,
                            ph=ph, pw=pw, pwin=pwin)
    return pl.pallas_call(
        kfn,
        grid=(B,),
        in_specs=[
            pl.BlockSpec((1, H, W, Ci), lambda b: (b, 0, 0, 0)),
            pl.BlockSpec((3, 3 * Ci, Co), lambda b: (0, 0, 0)),
            pl.BlockSpec((3, Co), lambda b: (0, 0)),
        ],
        out_specs=pl.BlockSpec((1, H // ph, Wp // pw, Co),
                               lambda b: (b, 0, 0, 0)),
        out_shape=jax.ShapeDtypeStruct((B, H // ph, Wp // pw, Co), F32),
        scratch_shapes=[pltpu.VMEM((H + 2, Wp, 3 * Ci), F32)],
        compiler_params=pltpu.CompilerParams(
            dimension_semantics=("parallel",),
            vmem_limit_bytes=110 * 1024 * 1024,
        ),
    )(x, w.reshape(3, 3 * Ci, Co), bss)


def _conv1_kernel(x_ref, w_ref, bss_ref, o_ref, x3_ref, *, H, W, Co):
    b_c = bss_ref[:, 0:1]
    s_c = bss_ref[:, 1:2]
    t_c = bss_ref[:, 2:3]
    zx1 = jnp.zeros((1, 1), F32)
    x3_ref[0] = jnp.zeros((3, W), F32)
    x3_ref[H + 1] = jnp.zeros((3, W), F32)

    def build(h, c):
        xr = x_ref[0, h]
        left = jnp.concatenate([zx1, xr[:, :-1]], axis=1)
        right = jnp.concatenate([xr[:, 1:], zx1], axis=1)
        x3_ref[h + 1] = jnp.concatenate([left, xr, right], axis=0)
        return c

    lax.fori_loop(0, H, build, 0)

    def consume(i, c):
        zs = []
        for r in range(2):
            cr = 2 * i + r
            acc = (jnp.dot(w_ref[0], x3_ref[cr], preferred_element_type=F32)
                   + jnp.dot(w_ref[1], x3_ref[cr + 1],
                             preferred_element_type=F32)
                   + jnp.dot(w_ref[2], x3_ref[cr + 2],
                             preferred_element_type=F32))
            zs.append(jnp.maximum(acc + b_c, 0.0) * s_c + t_c)
        o_ref[0, i] = jnp.maximum(zs[0], zs[1])
        return c

    lax.fori_loop(0, H // 2, consume, 0)


def _conv1_call(x, w, bss, *, H=128, W=512, Co=64, B=8):
    kfn = functools.partial(_conv1_kernel, H=H, W=W, Co=Co)
    return pl.pallas_call(
        kfn,
        grid=(B,),
        in_specs=[
            pl.BlockSpec((1, H, 1, W), lambda b: (b, 0, 0, 0)),
            pl.BlockSpec((3, Co, 3), lambda b: (0, 0, 0)),
            pl.BlockSpec((Co, 3), lambda b: (0, 0)),
        ],
        out_specs=pl.BlockSpec((1, H // 2, Co, W), lambda b: (b, 0, 0, 0)),
        out_shape=jax.ShapeDtypeStruct((B, H // 2, Co, W), F32),
        scratch_shapes=[pltpu.VMEM((H + 2, 3, W), F32)],
        compiler_params=pltpu.CompilerParams(
            dimension_semantics=("parallel",),
            vmem_limit_bytes=110 * 1024 * 1024,
        ),
    )(x, w.reshape(3, 3, Co).transpose(0, 2, 1), bss.T)


def _bss(p):
    scale = p["gamma"] * lax.rsqrt(p["var"] + _EPS)
    shift = p["beta"] - p["mean"] * scale
    return jnp.stack([p["b"], scale, shift])


def _rowgru_kernel(x_ref, pw_ref, pb_ref, W_ref, U_ref, b_ref, o_ref, gx_ref):
    d = pl.program_id(0)
    Wd = W_ref[d]
    Ud = U_ref[d]
    Wc = jnp.dot(pw_ref[...], Wd, preferred_element_type=F32)
    brow = jnp.dot(pb_ref[...], Wd, preferred_element_type=F32) + b_ref[d, 0:1]
    xflat = x_ref[...].reshape(64 * 128, 256)
    gx = jnp.dot(xflat, Wc, preferred_element_type=F32) + brow
    gx_ref[...] = gx.reshape(64, 128, 384)
    bh = b_ref[d, 1:2]

    def step(i, h):
        t = jnp.where(d == 0, i, 63 - i)
        gxt = gx_ref[t]
        gh = jnp.dot(h, Ud, preferred_element_type=F32) + bh
        z = jax.nn.sigmoid(gxt[:, :128] + gh[:, :128])
        r = jax.nn.sigmoid(gxt[:, 128:256] + gh[:, 128:256])
        cand = jnp.tanh(gxt[:, 256:] + r * gh[:, 256:])
        hn = z * h + (1.0 - z) * cand
        o_ref[0, t] = hn
        return hn

    lax.fori_loop(0, 64, step, jnp.zeros((128, 128), F32))


def _rowgru_call(xr, pw, pb, Wb, Ub, bb):
    return pl.pallas_call(
        _rowgru_kernel,
        grid=(2,),
        in_specs=[
            pl.BlockSpec((64, 128, 256), lambda d: (0, 0, 0)),
            pl.BlockSpec((256, 256), lambda d: (0, 0)),
            pl.BlockSpec((1, 256), lambda d: (0, 0)),
            pl.BlockSpec((2, 256, 384), lambda d: (0, 0, 0)),
            pl.BlockSpec((2, 128, 384), lambda d: (0, 0, 0)),
            pl.BlockSpec((2, 2, 384), lambda d: (0, 0, 0)),
        ],
        out_specs=pl.BlockSpec((1, 64, 128, 128), lambda d: (d, 0, 0, 0)),
        out_shape=jax.ShapeDtypeStruct((2, 64, 128, 128), F32),
        scratch_shapes=[pltpu.VMEM((64, 128, 384), F32)],
        compiler_params=pltpu.CompilerParams(
            dimension_semantics=("parallel",),
            vmem_limit_bytes=110 * 1024 * 1024,
        ),
    )(xr, pw, pb, Wb, Ub, bb)


def _decoder_kernel(encp_ref, encT_ref, onehot_ref, E_ref, W1T_ref, qb_ref,
                    W2_ref, V_ref, Wde_ref, Wdc_ref, db_ref, U_ref,
                    fc1h_ref, fc1c_ref, fc1b_ref, fc2_ref, fc2b_ref,
                    o_ref, w1ft_ref, etop_ref, gxe_ref):
    for b in range(8):
        w1ft_ref[b] = jnp.dot(W1T_ref[...], encT_ref[b],
                              preferred_element_type=F32)
    etop_ref[...] = jnp.dot(E_ref[...], Wde_ref[...],
                            preferred_element_type=F32)
    gxe = jnp.dot(onehot_ref[...], etop_ref[...],
                  preferred_element_type=F32) + db_ref[0:1]
    gxe_ref[...] = gxe.reshape(64, 8, 768)
    bh = db_ref[1:2]
    Vcol = V_ref[...]

    def step(t, h):
        q = jnp.dot(h, W2_ref[...], preferred_element_type=F32)
        qT = jnp.transpose(q) + qb_ref[...]
        srows = []
        for b in range(8):
            tmp = jnp.tanh(w1ft_ref[b] + qT[:, b:b + 1]) * Vcol
            srows.append(jnp.sum(tmp, axis=0, keepdims=True))
        scores = jnp.concatenate(srows, axis=0)
        mx = jnp.max(scores, axis=1, keepdims=True)
        e = jnp.exp(scores - mx)
        den = jnp.sum(e, axis=1, keepdims=True)
        crows = []
        for b in range(8):
            crows.append(jnp.dot(e[b:b + 1], encp_ref[b],
                                 preferred_element_type=F32))
        ctx = jnp.concatenate(crows, axis=0) / den
        gx = gxe_ref[t] + jnp.dot(ctx, Wdc_ref[...],
                                  preferred_element_type=F32)
        gh = jnp.dot(h, U_ref[...], preferred_element_type=F32) + bh
        z = jax.nn.sigmoid(gx[:, :256] + gh[:, :256])
        r = jax.nn.sigmoid(gx[:, 256:512] + gh[:, 256:512])
        cand = jnp.tanh(gx[:, 512:] + r * gh[:, 512:])
        hn = z * h + (1.0 - z) * cand
        pre = jnp.tanh(jnp.dot(hn, fc1h_ref[...], preferred_element_type=F32)
                       + jnp.dot(ctx, fc1c_ref[...], preferred_element_type=F32)
                       + fc1b_ref[...])
        o_ref[t] = (jnp.dot(pre, fc2_ref[...], preferred_element_type=F32)
                    + fc2b_ref[...])
        return hn

    lax.fori_loop(0, 64, step, jnp.zeros((8, 256), F32))


def _decoder_call(encp, encT, onehot, E, W1T, qb, W2, V, Wde, Wdc, db, U,
                  fc1h, fc1c, fc1b, fc2, fc2b):
    return pl.pallas_call(
        _decoder_kernel,
        out_shape=jax.ShapeDtypeStruct((64, 8, 1000), F32),
        scratch_shapes=[
            pltpu.VMEM((8, 256, 1024), F32),
            pltpu.VMEM((1000, 768), F32),
            pltpu.VMEM((64, 8, 768), F32),
        ],
        compiler_params=pltpu.CompilerParams(
            vmem_limit_bytes=120 * 1024 * 1024,
        ),
    )(encp, encT, onehot, E, W1T, qb, W2, V, Wde, Wdc, db, U,
      fc1h, fc1c, fc1b, fc2, fc2b)


def kernel(images, tgt_in, params):
    p = params
    B = images.shape[0]

    x = images.reshape(B, 128, 1, 512)
    x = _conv1_call(x, p["conv1"]["w"], _bss(p["conv1"]))
    x = x.transpose(0, 1, 3, 2)
    x = _conv_call(x, p["conv2"]["w"], _bss(p["conv2"]),
                   H=64, W=512, Ci=64, Co=128, ph=2, pw=2, pwin=2)
    x = _conv_call(x, p["conv3a"]["w"], _bss(p["conv3a"]),
                   H=32, W=128, Ci=128, Co=256, ph=1, pw=1)
    x = _conv_call(x, p["conv3b"]["w"], _bss(p["conv3b"]),
                   H=32, W=128, Ci=256, Co=256, ph=2, pw=1)
    x = _conv_call(x, p["conv4a"]["w"], _bss(p["conv4a"]),
                   H=16, W=128, Ci=256, Co=256, ph=1, pw=1)
    x = _conv_call(x, p["conv4b"]["w"], _bss(p["conv4b"]),
                   H=16, W=128, Ci=256, Co=256, ph=1, pw=2)

    xr = x.transpose(2, 0, 1, 3).reshape(64, 128, 256)
    Wb = jnp.stack([p["row_fwd"]["W"], p["row_bwd"]["W"]])
    Ub = jnp.stack([p["row_fwd"]["U"], p["row_bwd"]["U"]])
    bb = jnp.stack([p["row_fwd"]["b"], p["row_bwd"]["b"]])
    hs = _rowgru_call(xr, p["proj"]["w"], p["proj"]["b"].reshape(1, 256),
                      Wb, Ub, bb)
    enc_t = jnp.concatenate([hs[0], hs[1]], axis=-1)
    encp = enc_t.reshape(64, B, 16, 256).transpose(1, 2, 0, 3).reshape(B, 1024, 256)
    encT = encp.transpose(0, 2, 1)

    onehot = jax.nn.one_hot(tgt_in.T.reshape(-1), 1000, dtype=F32)
    qb = (p["attn_W1"]["b"] + p["attn_W2"]["b"]).reshape(256, 1)
    logits3 = _decoder_call(
        encp, encT, onehot, p["embed"],
        p["attn_W1"]["w"].T, qb, p["attn_W2"]["w"], p["attn_V"]["w"],
        p["dec_gru"]["W"][:128], p["dec_gru"]["W"][128:], p["dec_gru"]["b"],
        p["dec_gru"]["U"],
        p["fc1"]["w"][:256], p["fc1"]["w"][256:], p["fc1"]["b"].reshape(1, 256),
        p["fc2"]["w"], p["fc2"]["b"].reshape(1, 1000))
    return logits3.transpose(1, 0, 2)


# speedup vs baseline: 1.4967x; 1.1164x over previous
import functools

import jax
import jax.numpy as jnp
from jax import lax
from jax.experimental import pallas as pl
from jax.experimental.pallas import tpu as pltpu

F32 = jnp.float32
_EPS = 1e-3


def _conv_kernel(x_ref, w_ref, bss_ref, o_ref, xcat_ref, *,
                 H, W, Ci, Co, ph, pw, pwin):
    Wp = W // pwin
    BF = jnp.bfloat16
    zrow = jnp.zeros((1, Ci), F32)
    xcat_ref[0] = jnp.zeros((Wp, 3 * Ci), BF)
    xcat_ref[H + 1] = jnp.zeros((Wp, 3 * Ci), BF)

    def build(h, c):
        xr = x_ref[0, h]
        if pwin > 1:
            xr = jnp.max(xr.reshape(Wp, pwin, Ci), axis=1)
        up = jnp.concatenate([zrow, xr[:-1]], axis=0)
        dn = jnp.concatenate([xr[1:], zrow], axis=0)
        xcat_ref[h + 1] = jnp.concatenate([up, xr, dn], axis=1).astype(BF)
        return c

    lax.fori_loop(0, H, build, 0)

    def consume(j, c):
        for u in range(2):
            i = 2 * j + u
            zs = []
            for r in range(ph):
                cr = ph * i + r
                acc = (jnp.dot(xcat_ref[cr], w_ref[0],
                               preferred_element_type=F32)
                       + jnp.dot(xcat_ref[cr + 1], w_ref[1],
                                 preferred_element_type=F32)
                       + jnp.dot(xcat_ref[cr + 2], w_ref[2],
                                 preferred_element_type=F32))
                z = jnp.maximum(acc + bss_ref[0:1, :], 0.0)
                zs.append(z * bss_ref[1:2, :] + bss_ref[2:3, :])
            z = zs[0] if ph == 1 else jnp.maximum(zs[0], zs[1])
            if pw > 1:
                z = jnp.max(z.reshape(Wp // pw, pw, Co), axis=1)
            o_ref[0, i] = z
        return c

    lax.fori_loop(0, H // ph // 2, consume, 0)


def _conv_call(x, w, bss, *, H, W, Ci, Co, ph, pw, pwin=1, B=8):
    Wp = W // pwin
    kfn = functools.partial(_conv_kernel, H=H, W=W, Ci=Ci, Co=Co,
                            ph=ph, pw=pw, pwin=pwin)
    return pl.pallas_call(
        kfn,
        grid=(B,),
        in_specs=[
            pl.BlockSpec((1, H, W, Ci), lambda b: (b, 0, 0, 0)),
            pl.BlockSpec((3, 3 * Ci, Co), lambda b: (0, 0, 0)),
            pl.BlockSpec((3, Co), lambda b: (0, 0)),
        ],
        out_specs=pl.BlockSpec((1, H // ph, Wp // pw, Co),
                               lambda b: (b, 0, 0, 0)),
        out_shape=jax.ShapeDtypeStruct((B, H // ph, Wp // pw, Co), F32),
        scratch_shapes=[pltpu.VMEM((H + 2, Wp, 3 * Ci), jnp.bfloat16)],
        compiler_params=pltpu.CompilerParams(
            dimension_semantics=("parallel",),
            vmem_limit_bytes=110 * 1024 * 1024,
        ),
    )(x, w.reshape(3, 3 * Ci, Co).astype(jnp.bfloat16), bss)


def _conv1_kernel(x_ref, w_ref, bss_ref, o_ref, x3_ref, *, H, W, Co):
    b_c = bss_ref[:, 0:1]
    s_c = bss_ref[:, 1:2]
    t_c = bss_ref[:, 2:3]
    BF = jnp.bfloat16
    zx1 = jnp.zeros((1, 1), F32)
    x3_ref[0] = jnp.zeros((3, W), BF)
    x3_ref[H + 1] = jnp.zeros((3, W), BF)

    def build(h, c):
        xr = x_ref[0, h]
        left = jnp.concatenate([zx1, xr[:, :-1]], axis=1)
        right = jnp.concatenate([xr[:, 1:], zx1], axis=1)
        x3_ref[h + 1] = jnp.concatenate([left, xr, right], axis=0).astype(BF)
        return c

    lax.fori_loop(0, H, build, 0)

    def consume(j, c):
        for u in range(2):
            i = 2 * j + u
            zs = []
            for r in range(2):
                cr = 2 * i + r
                acc = (jnp.dot(w_ref[0], x3_ref[cr],
                               preferred_element_type=F32)
                       + jnp.dot(w_ref[1], x3_ref[cr + 1],
                                 preferred_element_type=F32)
                       + jnp.dot(w_ref[2], x3_ref[cr + 2],
                                 preferred_element_type=F32))
                zs.append(jnp.maximum(acc + b_c, 0.0) * s_c + t_c)
            o_ref[0, i] = jnp.maximum(zs[0], zs[1])
        return c

    lax.fori_loop(0, H // 4, consume, 0)


def _conv1_call(x, w, bss, *, H=128, W=512, Co=64, B=8):
    kfn = functools.partial(_conv1_kernel, H=H, W=W, Co=Co)
    return pl.pallas_call(
        kfn,
        grid=(B,),
        in_specs=[
            pl.BlockSpec((1, H, 1, W), lambda b: (b, 0, 0, 0)),
            pl.BlockSpec((3, Co, 3), lambda b: (0, 0, 0)),
            pl.BlockSpec((Co, 3), lambda b: (0, 0)),
        ],
        out_specs=pl.BlockSpec((1, H // 2, Co, W), lambda b: (b, 0, 0, 0)),
        out_shape=jax.ShapeDtypeStruct((B, H // 2, Co, W), F32),
        scratch_shapes=[pltpu.VMEM((H + 2, 3, W), jnp.bfloat16)],
        compiler_params=pltpu.CompilerParams(
            dimension_semantics=("parallel",),
            vmem_limit_bytes=110 * 1024 * 1024,
        ),
    )(x, w.reshape(3, 3, Co).transpose(0, 2, 1).astype(jnp.bfloat16), bss.T)


def _bss(p):
    scale = p["gamma"] * lax.rsqrt(p["var"] + _EPS)
    shift = p["beta"] - p["mean"] * scale
    return jnp.stack([p["b"], scale, shift])


def _rowgru_kernel(x_ref, pw_ref, pb_ref, W_ref, U_ref, b_ref, o_ref, gx_ref):
    d = pl.program_id(0)
    Wd = W_ref[d]
    Ud = U_ref[d]
    Wc = jnp.dot(pw_ref[...], Wd, preferred_element_type=F32)
    brow = jnp.dot(pb_ref[...], Wd, preferred_element_type=F32) + b_ref[d, 0:1]
    xflat = x_ref[...].reshape(64 * 128, 256)
    gx = jnp.dot(xflat, Wc, preferred_element_type=F32) + brow
    gx_ref[...] = gx.reshape(64, 128, 384)
    bh = b_ref[d, 1:2]

    def step(i, h):
        t = jnp.where(d == 0, i, 63 - i)
        gxt = gx_ref[t]
        gh = jnp.dot(h, Ud, preferred_element_type=F32) + bh
        z = jax.nn.sigmoid(gxt[:, :128] + gh[:, :128])
        r = jax.nn.sigmoid(gxt[:, 128:256] + gh[:, 128:256])
        cand = jnp.tanh(gxt[:, 256:] + r * gh[:, 256:])
        hn = z * h + (1.0 - z) * cand
        o_ref[0, t] = hn
        return hn

    lax.fori_loop(0, 64, step, jnp.zeros((128, 128), F32))


def _rowgru_call(xr, pw, pb, Wb, Ub, bb):
    return pl.pallas_call(
        _rowgru_kernel,
        grid=(2,),
        in_specs=[
            pl.BlockSpec((64, 128, 256), lambda d: (0, 0, 0)),
            pl.BlockSpec((256, 256), lambda d: (0, 0)),
            pl.BlockSpec((1, 256), lambda d: (0, 0)),
            pl.BlockSpec((2, 256, 384), lambda d: (0, 0, 0)),
            pl.BlockSpec((2, 128, 384), lambda d: (0, 0, 0)),
            pl.BlockSpec((2, 2, 384), lambda d: (0, 0, 0)),
        ],
        out_specs=pl.BlockSpec((1, 64, 128, 128), lambda d: (d, 0, 0, 0)),
        out_shape=jax.ShapeDtypeStruct((2, 64, 128, 128), F32),
        scratch_shapes=[pltpu.VMEM((64, 128, 384), F32)],
        compiler_params=pltpu.CompilerParams(
            dimension_semantics=("parallel",),
            vmem_limit_bytes=110 * 1024 * 1024,
        ),
    )(xr, pw, pb, Wb, Ub, bb)


def _decoder_kernel(encp_ref, encT_ref, onehot_ref, E_ref, W1T_ref, qb_ref,
                    W2_ref, V_ref, Wde_ref, Wdc_ref, db_ref, U_ref,
                    fc1h_ref, fc1c_ref, fc1b_ref, fc2_ref, fc2b_ref,
                    o_ref, w1ft_ref, etop_ref, gxe_ref):
    for b in range(8):
        w1ft_ref[b] = jnp.dot(W1T_ref[...], encT_ref[b],
                              preferred_element_type=F32)
    etop_ref[...] = jnp.dot(E_ref[...], Wde_ref[...],
                            preferred_element_type=F32)
    gxe = jnp.dot(onehot_ref[...], etop_ref[...],
                  preferred_element_type=F32) + db_ref[0:1]
    gxe_ref[...] = gxe.reshape(64, 8, 768)
    bh = db_ref[1:2]
    Vcol = V_ref[...]

    def step(t, h):
        q = jnp.dot(h, W2_ref[...], preferred_element_type=F32)
        qT = jnp.transpose(q) + qb_ref[...]
        srows = []
        for b in range(8):
            tmp = jnp.tanh(w1ft_ref[b] + qT[:, b:b + 1]) * Vcol
            srows.append(jnp.sum(tmp, axis=0, keepdims=True))
        scores = jnp.concatenate(srows, axis=0)
        mx = jnp.max(scores, axis=1, keepdims=True)
        e = jnp.exp(scores - mx)
        den = jnp.sum(e, axis=1, keepdims=True)
        crows = []
        for b in range(8):
            crows.append(jnp.dot(e[b:b + 1], encp_ref[b],
                                 preferred_element_type=F32))
        ctx = jnp.concatenate(crows, axis=0) / den
        gx = gxe_ref[t] + jnp.dot(ctx, Wdc_ref[...],
                                  preferred_element_type=F32)
        gh = jnp.dot(h, U_ref[...], preferred_element_type=F32) + bh
        z = jax.nn.sigmoid(gx[:, :256] + gh[:, :256])
        r = jax.nn.sigmoid(gx[:, 256:512] + gh[:, 256:512])
        cand = jnp.tanh(gx[:, 512:] + r * gh[:, 512:])
        hn = z * h + (1.0 - z) * cand
        pre = jnp.tanh(jnp.dot(hn, fc1h_ref[...], preferred_element_type=F32)
                       + jnp.dot(ctx, fc1c_ref[...], preferred_element_type=F32)
                       + fc1b_ref[...])
        o_ref[t] = (jnp.dot(pre, fc2_ref[...], preferred_element_type=F32)
                    + fc2b_ref[...])
        return hn

    lax.fori_loop(0, 64, step, jnp.zeros((8, 256), F32))


def _decoder_call(encp, encT, onehot, E, W1T, qb, W2, V, Wde, Wdc, db, U,
                  fc1h, fc1c, fc1b, fc2, fc2b):
    return pl.pallas_call(
        _decoder_kernel,
        out_shape=jax.ShapeDtypeStruct((64, 8, 1000), F32),
        scratch_shapes=[
            pltpu.VMEM((8, 256, 1024), F32),
            pltpu.VMEM((1000, 768), F32),
            pltpu.VMEM((64, 8, 768), F32),
        ],
        compiler_params=pltpu.CompilerParams(
            vmem_limit_bytes=120 * 1024 * 1024,
        ),
    )(encp, encT, onehot, E, W1T, qb, W2, V, Wde, Wdc, db, U,
      fc1h, fc1c, fc1b, fc2, fc2b)


def kernel(images, tgt_in, params):
    p = params
    B = images.shape[0]

    x = images.reshape(B, 128, 1, 512)
    x = _conv1_call(x, p["conv1"]["w"], _bss(p["conv1"]))
    x = x.transpose(0, 1, 3, 2)
    x = _conv_call(x, p["conv2"]["w"], _bss(p["conv2"]),
                   H=64, W=512, Ci=64, Co=128, ph=2, pw=2, pwin=2)
    x = _conv_call(x, p["conv3a"]["w"], _bss(p["conv3a"]),
                   H=32, W=128, Ci=128, Co=256, ph=1, pw=1)
    x = _conv_call(x, p["conv3b"]["w"], _bss(p["conv3b"]),
                   H=32, W=128, Ci=256, Co=256, ph=2, pw=1)
    x = _conv_call(x, p["conv4a"]["w"], _bss(p["conv4a"]),
                   H=16, W=128, Ci=256, Co=256, ph=1, pw=1)
    x = _conv_call(x, p["conv4b"]["w"], _bss(p["conv4b"]),
                   H=16, W=128, Ci=256, Co=256, ph=1, pw=2)

    xr = x.transpose(2, 0, 1, 3).reshape(64, 128, 256)
    Wb = jnp.stack([p["row_fwd"]["W"], p["row_bwd"]["W"]])
    Ub = jnp.stack([p["row_fwd"]["U"], p["row_bwd"]["U"]])
    bb = jnp.stack([p["row_fwd"]["b"], p["row_bwd"]["b"]])
    hs = _rowgru_call(xr, p["proj"]["w"], p["proj"]["b"].reshape(1, 256),
                      Wb, Ub, bb)
    enc_t = jnp.concatenate([hs[0], hs[1]], axis=-1)
    encp = enc_t.reshape(64, B, 16, 256).transpose(1, 2, 0, 3).reshape(B, 1024, 256)
    encT = encp.transpose(0, 2, 1)

    onehot = jax.nn.one_hot(tgt_in.T.reshape(-1), 1000, dtype=F32)
    qb = (p["attn_W1"]["b"] + p["attn_W2"]["b"]).reshape(256, 1)
    logits3 = _decoder_call(
        encp, encT, onehot, p["embed"],
        p["attn_W1"]["w"].T, qb, p["attn_W2"]["w"], p["attn_V"]["w"],
        p["dec_gru"]["W"][:128], p["dec_gru"]["W"][128:], p["dec_gru"]["b"],
        p["dec_gru"]["U"],
        p["fc1"]["w"][:256], p["fc1"]["w"][256:], p["fc1"]["b"].reshape(1, 256),
        p["fc2"]["w"], p["fc2"]["b"].reshape(1, 1000))
    return logits3.transpose(1, 0, 2)


# speedup vs baseline: 1.5430x; 1.0309x over previous
import functools

import jax
import jax.numpy as jnp
from jax import lax
from jax.experimental import pallas as pl
from jax.experimental.pallas import tpu as pltpu

F32 = jnp.float32
_EPS = 1e-3


def _conv_kernel(x_ref, w_ref, bss_ref, o_ref, xcat_ref, *,
                 H, W, Ci, Co, ph, pw, pwin):
    Wp = W // pwin
    BF = jnp.bfloat16
    zrow = jnp.zeros((1, Ci), F32)
    xcat_ref[0] = jnp.zeros((Wp, 3 * Ci), BF)
    xcat_ref[H + 1] = jnp.zeros((Wp, 3 * Ci), BF)

    def build(h, c):
        xr = x_ref[0, h]
        if pwin > 1:
            xr = jnp.max(xr.reshape(Wp, pwin, Ci), axis=1)
        up = jnp.concatenate([zrow, xr[:-1]], axis=0)
        dn = jnp.concatenate([xr[1:], zrow], axis=0)
        xcat_ref[h + 1] = jnp.concatenate([up, xr, dn], axis=1).astype(BF)
        return c

    lax.fori_loop(0, H, build, 0)

    def consume(j, c):
        cr0 = 8 * j
        acc = None
        for dy in range(3):
            xs = xcat_ref[pl.ds(cr0 + dy, 8)].reshape(8 * Wp, 3 * Ci)
            d = jnp.dot(xs, w_ref[dy], preferred_element_type=F32)
            acc = d if acc is None else acc + d
        z = jnp.maximum(acc + bss_ref[0:1, :], 0.0)
        z = z * bss_ref[1:2, :] + bss_ref[2:3, :]
        z = z.reshape(8 // ph, ph, Wp // pw, pw, Co)
        if ph == 2:
            z = jnp.max(z, axis=1)
        else:
            z = z.reshape(8, Wp // pw, pw, Co)
        if pw > 1:
            z = jnp.max(z, axis=-2)
        else:
            z = z.reshape(8 // ph, Wp, Co)
        o_ref[0, pl.ds((8 // ph) * j, 8 // ph)] = z
        return c

    lax.fori_loop(0, H // 8, consume, 0)


def _conv_call(x, w, bss, *, H, W, Ci, Co, ph, pw, pwin=1, B=8):
    Wp = W // pwin
    kfn = functools.partial(_conv_kernel, H=H, W=W, Ci=Ci, Co=Co,
                            ph=ph, pw=pw, pwin=pwin)
    return pl.pallas_call(
        kfn,
        grid=(B,),
        in_specs=[
            pl.BlockSpec((1, H, W, Ci), lambda b: (b, 0, 0, 0)),
            pl.BlockSpec((3, 3 * Ci, Co), lambda b: (0, 0, 0)),
            pl.BlockSpec((3, Co), lambda b: (0, 0)),
        ],
        out_specs=pl.BlockSpec((1, H // ph, Wp // pw, Co),
                               lambda b: (b, 0, 0, 0)),
        out_shape=jax.ShapeDtypeStruct((B, H // ph, Wp // pw, Co), F32),
        scratch_shapes=[pltpu.VMEM((H + 2, Wp, 3 * Ci), jnp.bfloat16)],
        compiler_params=pltpu.CompilerParams(
            dimension_semantics=("parallel",),
            vmem_limit_bytes=110 * 1024 * 1024,
        ),
    )(x, w.reshape(3, 3 * Ci, Co).astype(jnp.bfloat16), bss)


def _conv1_kernel(x_ref, w_ref, bss_ref, o_ref, x3_ref, *, H, W, Co):
    b_c = bss_ref[:, 0:1]
    s_c = bss_ref[:, 1:2]
    t_c = bss_ref[:, 2:3]
    BF = jnp.bfloat16
    zx1 = jnp.zeros((1, 1), F32)
    x3_ref[0] = jnp.zeros((3, W), BF)
    x3_ref[H + 1] = jnp.zeros((3, W), BF)

    def build(h, c):
        xr = x_ref[0, h]
        left = jnp.concatenate([zx1, xr[:, :-1]], axis=1)
        right = jnp.concatenate([xr[:, 1:], zx1], axis=1)
        x3_ref[h + 1] = jnp.concatenate([left, xr, right], axis=0).astype(BF)
        return c

    lax.fori_loop(0, H, build, 0)

    def consume(j, c):
        for u in range(2):
            i = 2 * j + u
            zs = []
            for r in range(2):
                cr = 2 * i + r
                acc = (jnp.dot(w_ref[0], x3_ref[cr],
                               preferred_element_type=F32)
                       + jnp.dot(w_ref[1], x3_ref[cr + 1],
                                 preferred_element_type=F32)
                       + jnp.dot(w_ref[2], x3_ref[cr + 2],
                                 preferred_element_type=F32))
                zs.append(jnp.maximum(acc + b_c, 0.0) * s_c + t_c)
            o_ref[0, i] = jnp.maximum(zs[0], zs[1])
        return c

    lax.fori_loop(0, H // 4, consume, 0)


def _conv1_call(x, w, bss, *, H=128, W=512, Co=64, B=8):
    kfn = functools.partial(_conv1_kernel, H=H, W=W, Co=Co)
    return pl.pallas_call(
        kfn,
        grid=(B,),
        in_specs=[
            pl.BlockSpec((1, H, 1, W), lambda b: (b, 0, 0, 0)),
            pl.BlockSpec((3, Co, 3), lambda b: (0, 0, 0)),
            pl.BlockSpec((Co, 3), lambda b: (0, 0)),
        ],
        out_specs=pl.BlockSpec((1, H // 2, Co, W), lambda b: (b, 0, 0, 0)),
        out_shape=jax.ShapeDtypeStruct((B, H // 2, Co, W), F32),
        scratch_shapes=[pltpu.VMEM((H + 2, 3, W), jnp.bfloat16)],
        compiler_params=pltpu.CompilerParams(
            dimension_semantics=("parallel",),
            vmem_limit_bytes=110 * 1024 * 1024,
        ),
    )(x, w.reshape(3, 3, Co).transpose(0, 2, 1).astype(jnp.bfloat16), bss.T)


def _bss(p):
    scale = p["gamma"] * lax.rsqrt(p["var"] + _EPS)
    shift = p["beta"] - p["mean"] * scale
    return jnp.stack([p["b"], scale, shift])


def _rowgru_kernel(x_ref, pw_ref, pb_ref, W_ref, U_ref, b_ref, o_ref, gx_ref):
    d = pl.program_id(0)
    Wd = W_ref[d]
    Ud = U_ref[d]
    Wc = jnp.dot(pw_ref[...], Wd, preferred_element_type=F32)
    brow = jnp.dot(pb_ref[...], Wd, preferred_element_type=F32) + b_ref[d, 0:1]
    xflat = x_ref[...].reshape(64 * 128, 256)
    gx = jnp.dot(xflat, Wc, preferred_element_type=F32) + brow
    gx_ref[...] = gx.reshape(64, 128, 384)
    bh = b_ref[d, 1:2]

    def step(i, h):
        t = jnp.where(d == 0, i, 63 - i)
        gxt = gx_ref[t]
        gh = jnp.dot(h, Ud, preferred_element_type=F32) + bh
        z = jax.nn.sigmoid(gxt[:, :128] + gh[:, :128])
        r = jax.nn.sigmoid(gxt[:, 128:256] + gh[:, 128:256])
        cand = jnp.tanh(gxt[:, 256:] + r * gh[:, 256:])
        hn = z * h + (1.0 - z) * cand
        o_ref[0, t] = hn
        return hn

    lax.fori_loop(0, 64, step, jnp.zeros((128, 128), F32))


def _rowgru_call(xr, pw, pb, Wb, Ub, bb):
    return pl.pallas_call(
        _rowgru_kernel,
        grid=(2,),
        in_specs=[
            pl.BlockSpec((64, 128, 256), lambda d: (0, 0, 0)),
            pl.BlockSpec((256, 256), lambda d: (0, 0)),
            pl.BlockSpec((1, 256), lambda d: (0, 0)),
            pl.BlockSpec((2, 256, 384), lambda d: (0, 0, 0)),
            pl.BlockSpec((2, 128, 384), lambda d: (0, 0, 0)),
            pl.BlockSpec((2, 2, 384), lambda d: (0, 0, 0)),
        ],
        out_specs=pl.BlockSpec((1, 64, 128, 128), lambda d: (d, 0, 0, 0)),
        out_shape=jax.ShapeDtypeStruct((2, 64, 128, 128), F32),
        scratch_shapes=[pltpu.VMEM((64, 128, 384), F32)],
        compiler_params=pltpu.CompilerParams(
            dimension_semantics=("parallel",),
            vmem_limit_bytes=110 * 1024 * 1024,
        ),
    )(xr, pw, pb, Wb, Ub, bb)


def _decoder_kernel(encp_ref, encT_ref, onehot_ref, E_ref, W1T_ref, qb_ref,
                    W2_ref, V_ref, Wde_ref, Wdc_ref, db_ref, U_ref,
                    fc1h_ref, fc1c_ref, fc1b_ref, fc2_ref, fc2b_ref,
                    o_ref, w1ft_ref, etop_ref, gxe_ref):
    for b in range(8):
        w1ft_ref[b] = jnp.dot(W1T_ref[...], encT_ref[b],
                              preferred_element_type=F32)
    etop_ref[...] = jnp.dot(E_ref[...], Wde_ref[...],
                            preferred_element_type=F32)
    gxe = jnp.dot(onehot_ref[...], etop_ref[...],
                  preferred_element_type=F32) + db_ref[0:1]
    gxe_ref[...] = gxe.reshape(64, 8, 768)
    bh = db_ref[1:2]
    Vcol = V_ref[...]

    def step(t, h):
        q = jnp.dot(h, W2_ref[...], preferred_element_type=F32)
        qT = jnp.transpose(q) + qb_ref[...]
        srows = []
        for b in range(8):
            tmp = jnp.tanh(w1ft_ref[b] + qT[:, b:b + 1]) * Vcol
            srows.append(jnp.sum(tmp, axis=0, keepdims=True))
        scores = jnp.concatenate(srows, axis=0)
        mx = jnp.max(scores, axis=1, keepdims=True)
        e = jnp.exp(scores - mx)
        den = jnp.sum(e, axis=1, keepdims=True)
        crows = []
        for b in range(8):
            crows.append(jnp.dot(e[b:b + 1], encp_ref[b],
                                 preferred_element_type=F32))
        ctx = jnp.concatenate(crows, axis=0) / den
        gx = gxe_ref[t] + jnp.dot(ctx, Wdc_ref[...],
                                  preferred_element_type=F32)
        gh = jnp.dot(h, U_ref[...], preferred_element_type=F32) + bh
        z = jax.nn.sigmoid(gx[:, :256] + gh[:, :256])
        r = jax.nn.sigmoid(gx[:, 256:512] + gh[:, 256:512])
        cand = jnp.tanh(gx[:, 512:] + r * gh[:, 512:])
        hn = z * h + (1.0 - z) * cand
        pre = jnp.tanh(jnp.dot(hn, fc1h_ref[...], preferred_element_type=F32)
                       + jnp.dot(ctx, fc1c_ref[...], preferred_element_type=F32)
                       + fc1b_ref[...])
        o_ref[t] = (jnp.dot(pre, fc2_ref[...], preferred_element_type=F32)
                    + fc2b_ref[...])
        return hn

    lax.fori_loop(0, 64, step, jnp.zeros((8, 256), F32))


def _decoder_call(encp, encT, onehot, E, W1T, qb, W2, V, Wde, Wdc, db, U,
                  fc1h, fc1c, fc1b, fc2, fc2b):
    return pl.pallas_call(
        _decoder_kernel,
        out_shape=jax.ShapeDtypeStruct((64, 8, 1000), F32),
        scratch_shapes=[
            pltpu.VMEM((8, 256, 1024), F32),
            pltpu.VMEM((1000, 768), F32),
            pltpu.VMEM((64, 8, 768), F32),
        ],
        compiler_params=pltpu.CompilerParams(
            vmem_limit_bytes=120 * 1024 * 1024,
        ),
    )(encp, encT, onehot, E, W1T, qb, W2, V, Wde, Wdc, db, U,
      fc1h, fc1c, fc1b, fc2, fc2b)


def kernel(images, tgt_in, params):
    p = params
    B = images.shape[0]

    x = images.reshape(B, 128, 1, 512)
    x = _conv1_call(x, p["conv1"]["w"], _bss(p["conv1"]))
    x = x.transpose(0, 1, 3, 2)
    x = _conv_call(x, p["conv2"]["w"], _bss(p["conv2"]),
                   H=64, W=512, Ci=64, Co=128, ph=2, pw=2, pwin=2)
    x = _conv_call(x, p["conv3a"]["w"], _bss(p["conv3a"]),
                   H=32, W=128, Ci=128, Co=256, ph=1, pw=1)
    x = _conv_call(x, p["conv3b"]["w"], _bss(p["conv3b"]),
                   H=32, W=128, Ci=256, Co=256, ph=2, pw=1)
    x = _conv_call(x, p["conv4a"]["w"], _bss(p["conv4a"]),
                   H=16, W=128, Ci=256, Co=256, ph=1, pw=1)
    x = _conv_call(x, p["conv4b"]["w"], _bss(p["conv4b"]),
                   H=16, W=128, Ci=256, Co=256, ph=1, pw=2)

    xr = x.transpose(2, 0, 1, 3).reshape(64, 128, 256)
    Wb = jnp.stack([p["row_fwd"]["W"], p["row_bwd"]["W"]])
    Ub = jnp.stack([p["row_fwd"]["U"], p["row_bwd"]["U"]])
    bb = jnp.stack([p["row_fwd"]["b"], p["row_bwd"]["b"]])
    hs = _rowgru_call(xr, p["proj"]["w"], p["proj"]["b"].reshape(1, 256),
                      Wb, Ub, bb)
    enc_t = jnp.concatenate([hs[0], hs[1]], axis=-1)
    encp = enc_t.reshape(64, B, 16, 256).transpose(1, 2, 0, 3).reshape(B, 1024, 256)
    encT = encp.transpose(0, 2, 1)

    onehot = jax.nn.one_hot(tgt_in.T.reshape(-1), 1000, dtype=F32)
    qb = (p["attn_W1"]["b"] + p["attn_W2"]["b"]).reshape(256, 1)
    logits3 = _decoder_call(
        encp, encT, onehot, p["embed"],
        p["attn_W1"]["w"].T, qb, p["attn_W2"]["w"], p["attn_V"]["w"],
        p["dec_gru"]["W"][:128], p["dec_gru"]["W"][128:], p["dec_gru"]["b"],
        p["dec_gru"]["U"],
        p["fc1"]["w"][:256], p["fc1"]["w"][256:], p["fc1"]["b"].reshape(1, 256),
        p["fc2"]["w"], p["fc2"]["b"].reshape(1, 1000))
    return logits3.transpose(1, 0, 2)


# speedup vs baseline: 1.5764x; 1.0217x over previous
import functools

import jax
import jax.numpy as jnp
from jax import lax
from jax.experimental import pallas as pl
from jax.experimental.pallas import tpu as pltpu

F32 = jnp.float32
_EPS = 1e-3


def _conv_kernel(x_ref, w_ref, bss_ref, o_ref, xcat_ref, *,
                 H, W, Ci, Co, ph, pw, pwin):
    Wp = W // pwin
    BF = jnp.bfloat16
    zrow = jnp.zeros((1, Ci), F32)
    xcat_ref[0] = jnp.zeros((Wp, 3 * Ci), BF)
    xcat_ref[H + 1] = jnp.zeros((Wp, 3 * Ci), BF)

    def build(h, c):
        xr = x_ref[0, h]
        if pwin > 1:
            xr = jnp.max(xr.reshape(Wp, pwin, Ci), axis=1)
        up = jnp.concatenate([zrow, xr[:-1]], axis=0)
        dn = jnp.concatenate([xr[1:], zrow], axis=0)
        xcat_ref[h + 1] = jnp.concatenate([up, xr, dn], axis=1).astype(BF)
        return c

    lax.fori_loop(0, H, build, 0)

    def consume(j, c):
        cr0 = 8 * j
        acc = None
        for dy in range(3):
            xs = xcat_ref[pl.ds(cr0 + dy, 8)].reshape(8 * Wp, 3 * Ci)
            d = jnp.dot(xs, w_ref[dy], preferred_element_type=F32)
            acc = d if acc is None else acc + d
        z = jnp.maximum(acc + bss_ref[0:1, :], 0.0)
        z = z * bss_ref[1:2, :] + bss_ref[2:3, :]
        if ph == 2:
            z = jnp.max(z.reshape(4, 2, Wp, Co), axis=1)
        else:
            z = z.reshape(8, Wp, Co)
        o_ref[0, pl.ds((8 // ph) * j, 8 // ph)] = z
        return c

    lax.fori_loop(0, H // 8, consume, 0)


def _conv_call(x, w, bss, *, H, W, Ci, Co, ph, pw=1, pwin=1, B=8):
    Wp = W // pwin
    kfn = functools.partial(_conv_kernel, H=H, W=W, Ci=Ci, Co=Co,
                            ph=ph, pw=pw, pwin=pwin)
    return pl.pallas_call(
        kfn,
        grid=(B,),
        in_specs=[
            pl.BlockSpec((1, H, W, Ci), lambda b: (b, 0, 0, 0)),
            pl.BlockSpec((3, 3 * Ci, Co), lambda b: (0, 0, 0)),
            pl.BlockSpec((3, Co), lambda b: (0, 0)),
        ],
        out_specs=pl.BlockSpec((1, H // ph, Wp // pw, Co),
                               lambda b: (b, 0, 0, 0)),
        out_shape=jax.ShapeDtypeStruct((B, H // ph, Wp // pw, Co), F32),
        scratch_shapes=[pltpu.VMEM((H + 2, Wp, 3 * Ci), jnp.bfloat16)],
        compiler_params=pltpu.CompilerParams(
            dimension_semantics=("parallel",),
            vmem_limit_bytes=110 * 1024 * 1024,
        ),
    )(x, w.reshape(3, 3 * Ci, Co).astype(jnp.bfloat16), bss)


def _conv1_kernel(x_ref, w_ref, bss_ref, o_ref, x3_ref, *, H, W, Co):
    b_c = bss_ref[:, 0:1]
    s_c = bss_ref[:, 1:2]
    t_c = bss_ref[:, 2:3]
    BF = jnp.bfloat16
    zx1 = jnp.zeros((1, 1), F32)
    x3_ref[0] = jnp.zeros((3, W), BF)
    x3_ref[H + 1] = jnp.zeros((3, W), BF)

    def build(h, c):
        xr = x_ref[0, h]
        left = jnp.concatenate([zx1, xr[:, :-1]], axis=1)
        right = jnp.concatenate([xr[:, 1:], zx1], axis=1)
        x3_ref[h + 1] = jnp.concatenate([left, xr, right], axis=0).astype(BF)
        return c

    lax.fori_loop(0, H, build, 0)

    def consume(j, c):
        for u in range(2):
            i = 2 * j + u
            zs = []
            for r in range(2):
                cr = 2 * i + r
                acc = (jnp.dot(w_ref[0], x3_ref[cr],
                               preferred_element_type=F32)
                       + jnp.dot(w_ref[1], x3_ref[cr + 1],
                                 preferred_element_type=F32)
                       + jnp.dot(w_ref[2], x3_ref[cr + 2],
                                 preferred_element_type=F32))
                zs.append(jnp.maximum(acc + b_c, 0.0) * s_c + t_c)
            o_ref[0, i] = jnp.maximum(zs[0], zs[1])
        return c

    lax.fori_loop(0, H // 4, consume, 0)


def _conv1_call(x, w, bss, *, H=128, W=512, Co=64, B=8):
    kfn = functools.partial(_conv1_kernel, H=H, W=W, Co=Co)
    return pl.pallas_call(
        kfn,
        grid=(B,),
        in_specs=[
            pl.BlockSpec((1, H, 1, W), lambda b: (b, 0, 0, 0)),
            pl.BlockSpec((3, Co, 3), lambda b: (0, 0, 0)),
            pl.BlockSpec((Co, 3), lambda b: (0, 0)),
        ],
        out_specs=pl.BlockSpec((1, H // 2, Co, W), lambda b: (b, 0, 0, 0)),
        out_shape=jax.ShapeDtypeStruct((B, H // 2, Co, W), F32),
        scratch_shapes=[pltpu.VMEM((H + 2, 3, W), jnp.bfloat16)],
        compiler_params=pltpu.CompilerParams(
            dimension_semantics=("parallel",),
            vmem_limit_bytes=110 * 1024 * 1024,
        ),
    )(x, w.reshape(3, 3, Co).transpose(0, 2, 1).astype(jnp.bfloat16), bss.T)


def _bss(p):
    scale = p["gamma"] * lax.rsqrt(p["var"] + _EPS)
    shift = p["beta"] - p["mean"] * scale
    return jnp.stack([p["b"], scale, shift])


def _rowgru_kernel(x_ref, pw_ref, pb_ref, W_ref, U_ref, b_ref, o_ref, gx_ref):
    d = pl.program_id(0)
    Wd = W_ref[d]
    Ud = U_ref[d]
    Wc = jnp.dot(pw_ref[...], Wd, preferred_element_type=F32)
    brow = jnp.dot(pb_ref[...], Wd, preferred_element_type=F32) + b_ref[d, 0:1]
    xflat = jnp.max(x_ref[...], axis=1).reshape(64 * 128, 256)
    gx = jnp.dot(xflat, Wc, preferred_element_type=F32) + brow
    gx_ref[...] = gx.reshape(64, 128, 384)
    bh = b_ref[d, 1:2]

    def step(i, h):
        t = jnp.where(d == 0, i, 63 - i)
        gxt = gx_ref[t]
        gh = jnp.dot(h, Ud, preferred_element_type=F32) + bh
        z = jax.nn.sigmoid(gxt[:, :128] + gh[:, :128])
        r = jax.nn.sigmoid(gxt[:, 128:256] + gh[:, 128:256])
        cand = jnp.tanh(gxt[:, 256:] + r * gh[:, 256:])
        hn = z * h + (1.0 - z) * cand
        o_ref[0, t] = hn
        return hn

    lax.fori_loop(0, 64, step, jnp.zeros((128, 128), F32))


def _rowgru_call(xr, pw, pb, Wb, Ub, bb):
    return pl.pallas_call(
        _rowgru_kernel,
        grid=(2,),
        in_specs=[
            pl.BlockSpec((64, 2, 128, 256), lambda d: (0, 0, 0, 0)),
            pl.BlockSpec((256, 256), lambda d: (0, 0)),
            pl.BlockSpec((1, 256), lambda d: (0, 0)),
            pl.BlockSpec((2, 256, 384), lambda d: (0, 0, 0)),
            pl.BlockSpec((2, 128, 384), lambda d: (0, 0, 0)),
            pl.BlockSpec((2, 2, 384), lambda d: (0, 0, 0)),
        ],
        out_specs=pl.BlockSpec((1, 64, 128, 128), lambda d: (d, 0, 0, 0)),
        out_shape=jax.ShapeDtypeStruct((2, 64, 128, 128), F32),
        scratch_shapes=[pltpu.VMEM((64, 128, 384), F32)],
        compiler_params=pltpu.CompilerParams(
            dimension_semantics=("parallel",),
            vmem_limit_bytes=110 * 1024 * 1024,
        ),
    )(xr, pw, pb, Wb, Ub, bb)


def _decoder_kernel(encp_ref, encT_ref, onehot_ref, E_ref, W1T_ref, qb_ref,
                    W2_ref, V_ref, Wde_ref, Wdc_ref, db_ref, U_ref,
                    fc1h_ref, fc1c_ref, fc1b_ref, fc2_ref, fc2b_ref,
                    o_ref, w1ft_ref, etop_ref, gxe_ref):
    for b in range(8):
        w1ft_ref[b] = jnp.dot(W1T_ref[...], encT_ref[b],
                              preferred_element_type=F32)
    etop_ref[...] = jnp.dot(E_ref[...], Wde_ref[...],
                            preferred_element_type=F32)
    gxe = jnp.dot(onehot_ref[...], etop_ref[...],
                  preferred_element_type=F32) + db_ref[0:1]
    gxe_ref[...] = gxe.reshape(64, 8, 768)
    bh = db_ref[1:2]
    Vcol = V_ref[...]

    def step(t, h):
        q = jnp.dot(h, W2_ref[...], preferred_element_type=F32)
        qT = jnp.transpose(q) + qb_ref[...]
        srows = []
        for b in range(8):
            tmp = jnp.tanh(w1ft_ref[b] + qT[:, b:b + 1]) * Vcol
            srows.append(jnp.sum(tmp, axis=0, keepdims=True))
        scores = jnp.concatenate(srows, axis=0)
        mx = jnp.max(scores, axis=1, keepdims=True)
        e = jnp.exp(scores - mx)
        den = jnp.sum(e, axis=1, keepdims=True)
        crows = []
        for b in range(8):
            crows.append(jnp.dot(e[b:b + 1], encp_ref[b],
                                 preferred_element_type=F32))
        ctx = jnp.concatenate(crows, axis=0) / den
        gx = gxe_ref[t] + jnp.dot(ctx, Wdc_ref[...],
                                  preferred_element_type=F32)
        gh = jnp.dot(h, U_ref[...], preferred_element_type=F32) + bh
        z = jax.nn.sigmoid(gx[:, :256] + gh[:, :256])
        r = jax.nn.sigmoid(gx[:, 256:512] + gh[:, 256:512])
        cand = jnp.tanh(gx[:, 512:] + r * gh[:, 512:])
        hn = z * h + (1.0 - z) * cand
        pre = jnp.tanh(jnp.dot(hn, fc1h_ref[...], preferred_element_type=F32)
                       + jnp.dot(ctx, fc1c_ref[...], preferred_element_type=F32)
                       + fc1b_ref[...])
        o_ref[t] = (jnp.dot(pre, fc2_ref[...], preferred_element_type=F32)
                    + fc2b_ref[...])
        return hn

    lax.fori_loop(0, 64, step, jnp.zeros((8, 256), F32))


def _decoder_call(encp, encT, onehot, E, W1T, qb, W2, V, Wde, Wdc, db, U,
                  fc1h, fc1c, fc1b, fc2, fc2b):
    return pl.pallas_call(
        _decoder_kernel,
        out_shape=jax.ShapeDtypeStruct((64, 8, 1000), F32),
        scratch_shapes=[
            pltpu.VMEM((8, 256, 1024), F32),
            pltpu.VMEM((1000, 768), F32),
            pltpu.VMEM((64, 8, 768), F32),
        ],
        compiler_params=pltpu.CompilerParams(
            vmem_limit_bytes=120 * 1024 * 1024,
        ),
    )(encp, encT, onehot, E, W1T, qb, W2, V, Wde, Wdc, db, U,
      fc1h, fc1c, fc1b, fc2, fc2b)


def kernel(images, tgt_in, params):
    p = params
    B = images.shape[0]

    x = images.reshape(B, 128, 1, 512)
    x = _conv1_call(x, p["conv1"]["w"], _bss(p["conv1"]))
    x = x.transpose(0, 1, 3, 2)
    x = _conv_call(x, p["conv2"]["w"], _bss(p["conv2"]),
                   H=64, W=512, Ci=64, Co=128, ph=2, pwin=2)
    x = _conv_call(x, p["conv3a"]["w"], _bss(p["conv3a"]),
                   H=32, W=256, Ci=128, Co=256, ph=1, pwin=2)
    x = _conv_call(x, p["conv3b"]["w"], _bss(p["conv3b"]),
                   H=32, W=128, Ci=256, Co=256, ph=2)
    x = _conv_call(x, p["conv4a"]["w"], _bss(p["conv4a"]),
                   H=16, W=128, Ci=256, Co=256, ph=1)
    x = _conv_call(x, p["conv4b"]["w"], _bss(p["conv4b"]),
                   H=16, W=128, Ci=256, Co=256, ph=1)

    xr = x.transpose(2, 0, 1, 3).reshape(64, 2, 128, 256)
    Wb = jnp.stack([p["row_fwd"]["W"], p["row_bwd"]["W"]])
    Ub = jnp.stack([p["row_fwd"]["U"], p["row_bwd"]["U"]])
    bb = jnp.stack([p["row_fwd"]["b"], p["row_bwd"]["b"]])
    hs = _rowgru_call(xr, p["proj"]["w"], p["proj"]["b"].reshape(1, 256),
                      Wb, Ub, bb)
    enc_t = jnp.concatenate([hs[0], hs[1]], axis=-1)
    encp = enc_t.reshape(64, B, 16, 256).transpose(1, 2, 0, 3).reshape(B, 1024, 256)
    encT = encp.transpose(0, 2, 1)

    onehot = jax.nn.one_hot(tgt_in.T.reshape(-1), 1000, dtype=F32)
    qb = (p["attn_W1"]["b"] + p["attn_W2"]["b"]).reshape(256, 1)
    logits3 = _decoder_call(
        encp, encT, onehot, p["embed"],
        p["attn_W1"]["w"].T, qb, p["attn_W2"]["w"], p["attn_V"]["w"],
        p["dec_gru"]["W"][:128], p["dec_gru"]["W"][128:], p["dec_gru"]["b"],
        p["dec_gru"]["U"],
        p["fc1"]["w"][:256], p["fc1"]["w"][256:], p["fc1"]["b"].reshape(1, 256),
        p["fc2"]["w"], p["fc2"]["b"].reshape(1, 1000))
    return logits3.transpose(1, 0, 2)
